# baseline (speedup 1.0000x reference)
import functools
import jax
import jax.numpy as jnp
from jax import lax
from jax.experimental import pallas as pl
from jax.experimental.pallas import tpu as pltpu
from jax.experimental.pallas import tpu_sc as plsc

H = 128
NUM_F = 8
CAT_F = 6
BIN_F = 6
TOKENS = NUM_F + CAT_F + BIN_F
GATHER_F = CAT_F + BIN_F
CH = 8
NBUF = 3
L = 16
NJ = H // L
VOCABS = (151, 101, 21, 5, 4, 15)
NROWS_TAB = sum(VOCABS) + 2 * BIN_F


def kernel(numeric, categorical, binary, W_num, b_num, bin_emb,
           cat_emb_0, cat_emb_1, cat_emb_2, cat_emb_3, cat_emb_4, cat_emb_5):
    B = numeric.shape[0]
    info = plsc.get_sparse_core_info()
    NC, NS = info.num_cores, info.num_subcores
    NW = NC * NS
    rows_w = B // NW
    nch = rows_w // CH

    num_cp = (numeric.reshape(NW, nch, CH, NUM_F)
              .transpose(0, 1, 3, 2))
    num_cp = jnp.pad(num_cp, ((0, 0), (0, 0), (0, 0), (0, L - CH)))
    num_cp = num_cp.reshape(NW, nch * NUM_F * L)

    tabs_all = jnp.concatenate(
        [cat_emb_0, cat_emb_1, cat_emb_2, cat_emb_3, cat_emb_4, cat_emb_5,
         bin_emb.reshape(2 * BIN_F, H)], axis=0)
    cat_offs = [0]
    for v in VOCABS:
        cat_offs.append(cat_offs[-1] + v)
    coffs = jnp.asarray(cat_offs[:CAT_F], jnp.int32)[:, None]
    boffs = (cat_offs[CAT_F]
             + 2 * jnp.arange(BIN_F, dtype=jnp.int32))[:, None]
    idx_fm = jnp.concatenate(
        [categorical.T.astype(jnp.int32) + coffs,
         binary.T.astype(jnp.int32) + boffs], axis=0)
    idx_fm = (idx_fm.reshape(GATHER_F, NW, rows_w)
              .transpose(1, 0, 2))

    mesh = plsc.VectorSubcoreMesh(core_axis_name="c", subcore_axis_name="s")

    @functools.partial(
        pl.kernel, mesh=mesh,
        out_type=jax.ShapeDtypeStruct((B, TOKENS, H), jnp.float32),
        scratch_types=[
            pltpu.VMEM((NBUF, TOKENS, CH, H), jnp.float32),
            pltpu.VMEM((nch * NUM_F * L,), jnp.float32),
            pltpu.VMEM((GATHER_F, rows_w), jnp.int32),
            pltpu.VMEM((NUM_F, H), jnp.float32),
            pltpu.VMEM((NUM_F, H), jnp.float32),
            pltpu.SemaphoreType.DMA,
            pltpu.SemaphoreType.DMA,
        ],
    )
    def sck(num_hbm, idx_hbm, wn_hbm, bn_hbm, tab_hbm,
            out_hbm, staging, num_v, idx_v, w_v, b_v, gsem, wsem):
        wid = lax.axis_index("s") * NC + lax.axis_index("c")
        base = wid * rows_w
        pltpu.sync_copy(num_hbm.at[wid], num_v)
        pltpu.sync_copy(idx_hbm.at[wid], idx_v)
        pltpu.sync_copy(wn_hbm, w_v)
        pltpu.sync_copy(bn_hbm, b_v)

        def issue_gathers(cc):
            smod = cc % NBUF
            for i in range(GATHER_F):
                pltpu.async_copy(
                    tab_hbm.at[idx_v.at[i, pl.ds(cc * CH, CH)]],
                    staging.at[smod, NUM_F + i], gsem)

        def wait_gathers():
            for _ in range(GATHER_F):
                pltpu.make_async_copy(
                    tab_hbm.at[pl.ds(0, CH)], staging.at[0, NUM_F], gsem
                ).wait()

        def issue_writes(cc):
            smod = cc % NBUF
            for t in range(TOKENS):
                pltpu.async_copy(
                    staging.at[smod, t],
                    out_hbm.at[pl.ds(base + cc * CH, CH), t], wsem)

        def wait_writes():
            for _ in range(TOKENS):
                pltpu.make_async_copy(
                    staging.at[0, 0], out_hbm.at[pl.ds(base, CH), 0], wsem
                ).wait()

        issue_gathers(0)

        def do_chunk(c, _):
            smod = c % NBUF

            @pl.when(c >= 2)
            def _drain():
                wait_writes()

            @pl.when(c + 1 < nch)
            def _prefetch():
                issue_gathers(c + 1)

            xrows = [num_v[pl.ds(c * (NUM_F * L) + t * L, L)]
                     for t in range(NUM_F)]
            for t in range(NUM_F):
                wvecs = [w_v[t, pl.ds(L * j, L)] for j in range(NJ)]
                bvecs = [b_v[t, pl.ds(L * j, L)] for j in range(NJ)]
                for rl in range(CH):
                    xsp = jnp.full((L,), xrows[t][rl], jnp.float32)
                    for j in range(NJ):
                        staging[smod, t, rl, pl.ds(L * j, L)] = (
                            xsp * wvecs[j] + bvecs[j])

            wait_gathers()
            issue_writes(c)
            return 0

        lax.fori_loop(0, nch, do_chunk, 0)
        for _ in range(2):
            wait_writes()

    return sck(num_cp, idx_fm, W_num, b_num, tabs_all)
